# baseline (speedup 1.0000x reference)
import functools

import jax
import jax.numpy as jnp
from jax import lax
from jax.experimental import pallas as pl
from jax.experimental.pallas import tpu as pltpu
from jax.experimental.pallas import tpu_sc as plsc

TOPK = 2
SCALE = 1.0

TB_R = 256


def _pack_pair(v32):
    h2 = v32.shape[1] // 2
    vi = lax.bitcast_convert_type(v32, jnp.int32) + 32768
    return ((vi[:, :h2] >> 16) & 65535) | (vi[:, h2:] & (-65536))


def _unpack_pair(w):
    lo = lax.bitcast_convert_type(w << 16, jnp.float32)
    hi = lax.bitcast_convert_type(w & (-65536), jnp.float32)
    return lo, hi


def _router_body(x_ref, rw_ref, b_ref, sgw_ref, sdw_ref,
                 pr1_ref, pr2_ref, e1_ref, e2_ref, w1c_ref, w2c_ref,
                 xpk_ref, sh_ref, offs_ref, hist_ref, *, n_experts):
    i = pl.program_id(0)
    e = n_experts

    @pl.when(i == 0)
    def _():
        hist_ref[...] = jnp.zeros((1, e), jnp.float32)

    x = x_ref[...]
    logits = lax.dot_general(x, rw_ref[...], (((1,), (1,)), ((), ())),
                             preferred_element_type=jnp.float32) + b_ref[...]
    probs = jax.nn.sigmoid(logits)
    iota = lax.broadcasted_iota(jnp.int32, (TB_R, e), 1)
    m1 = jnp.max(probs, axis=1, keepdims=True)
    idx1 = jnp.min(jnp.where(probs >= m1, iota, e), axis=1, keepdims=True)
    sel1 = iota == idx1
    probs2 = jnp.where(sel1, -jnp.inf, probs)
    m2 = jnp.max(probs2, axis=1, keepdims=True)
    idx2 = jnp.min(jnp.where(probs2 >= m2, iota, e), axis=1, keepdims=True)
    sel2 = iota == idx2
    denom = m1 + m2 + 1e-9
    w1 = m1 / denom * SCALE
    w2 = m2 / denom * SCALE

    oh = jnp.where(sel1 | sel2, 1.0, 0.0)
    r_i = lax.broadcasted_iota(jnp.int32, (TB_R, TB_R), 0)
    c_i = lax.broadcasted_iota(jnp.int32, (TB_R, TB_R), 1)
    lstrict = jnp.where(c_i < r_i, 1.0, 0.0)
    excl = lax.dot_general(lstrict, oh, (((1,), (0,)), ((), ())),
                           preferred_element_type=jnp.float32,
                           precision=lax.Precision.HIGHEST)
    excl = excl + hist_ref[...]
    pr1 = jnp.sum(excl * jnp.where(sel1, 1.0, 0.0), axis=1, keepdims=True)
    pr2 = jnp.sum(excl * jnp.where(sel2, 1.0, 0.0), axis=1, keepdims=True)

    hist_new = hist_ref[...] + jnp.sum(oh, axis=0, keepdims=True)
    hist_ref[...] = hist_new

    re_i = lax.broadcasted_iota(jnp.int32, (e, e), 0)
    ce_i = lax.broadcasted_iota(jnp.int32, (e, e), 1)
    le = jnp.where(re_i < ce_i, 1.0, 0.0)
    offs = lax.dot_general(hist_new, le, (((1,), (0,)), ((), ())),
                           preferred_element_type=jnp.float32,
                           precision=lax.Precision.HIGHEST)
    offs_ref[...] = offs.reshape(1, 1, e)

    ident = jnp.where(r_i[:128, :128] == c_i[:128, :128], 1.0, 0.0)

    def t(col):
        top = lax.dot_general(col[0:128, :], ident, (((0,), (0,)), ((), ())),
                              preferred_element_type=jnp.float32,
                              precision=lax.Precision.HIGHEST)
        bot = lax.dot_general(col[128:256, :], ident, (((0,), (0,)), ((), ())),
                              preferred_element_type=jnp.float32,
                              precision=lax.Precision.HIGHEST)
        return jnp.concatenate([top, bot], axis=0).reshape(1, 2, 128)

    pr1_ref[...] = t(pr1)
    pr2_ref[...] = t(pr2)
    e1_ref[...] = t(idx1.astype(jnp.float32))
    e2_ref[...] = t(idx2.astype(jnp.float32))
    w1c_ref[...] = w1
    w2c_ref[...] = w2
    xpk_ref[...] = _pack_pair(x)
    sh_ref[...] = _ffn_bf16(x.astype(jnp.bfloat16), sgw_ref[...],
                            sdw_ref[...])


def _run_router(x, router_w, bias2d, sgw_bf, sdw_bf, t, h, f, e):
    nb = t // TB_R
    meta_shape = jax.ShapeDtypeStruct((t // 256, 2, 128), jnp.float32)
    body = functools.partial(_router_body, n_experts=e)
    cmeta = pl.BlockSpec((1, 2, 128), lambda i: (i, 0, 0))
    return pl.pallas_call(
        body,
        grid=(nb,),
        in_specs=[
            pl.BlockSpec((TB_R, h), lambda i: (i, 0)),
            pl.BlockSpec((e, h), lambda i: (0, 0)),
            pl.BlockSpec((1, e), lambda i: (0, 0)),
            pl.BlockSpec((f, h), lambda i: (0, 0)),
            pl.BlockSpec((h, f), lambda i: (0, 0)),
        ],
        out_specs=[cmeta, cmeta, cmeta, cmeta,
                   pl.BlockSpec((TB_R, 1), lambda i: (i, 0)),
                   pl.BlockSpec((TB_R, 1), lambda i: (i, 0)),
                   pl.BlockSpec((TB_R, h // 2), lambda i: (i, 0)),
                   pl.BlockSpec((TB_R, h), lambda i: (i, 0)),
                   pl.BlockSpec((1, 1, e), lambda i: (0, 0, 0))],
        out_shape=[meta_shape] * 4 + [
            jax.ShapeDtypeStruct((t, 1), jnp.float32)] * 2 + [
            jax.ShapeDtypeStruct((t, h // 2), jnp.int32),
            jax.ShapeDtypeStruct((t, h), jnp.float32),
            jax.ShapeDtypeStruct((1, 1, e), jnp.float32)],
        scratch_shapes=[pltpu.VMEM((1, e), jnp.float32)],
        compiler_params=pltpu.CompilerParams(
            dimension_semantics=("arbitrary",)),
    )(x, router_w, bias2d, sgw_bf, sdw_bf)


def _ffn_bf16(xb, gw, dw):
    hp = lax.dot_general(xb, gw, (((1,), (1,)), ((), ())),
                         preferred_element_type=jnp.float32)
    a = hp * jax.nn.sigmoid(hp)
    return lax.dot_general(a.astype(jnp.bfloat16), dw,
                           (((1,), (1,)), ((), ())),
                           preferred_element_type=jnp.float32)


def _shared_body(x_ref, sgw_ref, sdw_ref, o_ref):
    o_ref[...] = _ffn_bf16(x_ref[...].astype(jnp.bfloat16),
                           sgw_ref[...], sdw_ref[...])


def _run_shared(x, sgw_bf, sdw_bf, t, h, f):
    tb = 512
    return pl.pallas_call(
        _shared_body,
        grid=(t // tb,),
        in_specs=[
            pl.BlockSpec((tb, h), lambda i: (i, 0)),
            pl.BlockSpec((f, h), lambda i: (0, 0)),
            pl.BlockSpec((h, f), lambda i: (0, 0)),
        ],
        out_specs=pl.BlockSpec((tb, h), lambda i: (i, 0)),
        out_shape=jax.ShapeDtypeStruct((t, h), jnp.float32),
        compiler_params=pltpu.CompilerParams(
            dimension_semantics=("arbitrary",)),
    )(x, sgw_bf, sdw_bf)


def _load_meta_row(hbm, wid, dst):
    pltpu.sync_copy(hbm.at[wid // 2, wid % 2], dst)


def _pos_chunk(pr_v, e_v, offs_v, idx_v, j, nchunk):
    for c in range(nchunk // 16):
        sl = pl.ds(j * nchunk + c * 16, 16)
        eidx = e_v[sl].astype(jnp.int32)
        ofs = plsc.load_gather(offs_v, [eidx])
        idx_v[pl.ds(c * 16, 16)] = (pr_v[sl] + ofs + 0.5).astype(jnp.int32)


def _dispatch_body(x_hbm, pr1_hbm, pr2_hbm, e1_hbm, e2_hbm, offs_hbm,
                   xs_hbm,
                   offs_v, pr1_v, pr2_v, e1_v, e2_v, idx1_v, idx2_v,
                   rows_v, sem, *, tok_per_w, nchunk):
    wid = lax.axis_index("s") * 2 + lax.axis_index("c")
    base = wid * tok_per_w
    pltpu.sync_copy(offs_hbm, offs_v)
    _load_meta_row(pr1_hbm, wid, pr1_v)
    _load_meta_row(pr2_hbm, wid, pr2_v)
    _load_meta_row(e1_hbm, wid, e1_v)
    _load_meta_row(e2_hbm, wid, e2_v)
    for j in range(tok_per_w // nchunk):
        pltpu.sync_copy(x_hbm.at[pl.ds(base + j * nchunk, nchunk)], rows_v)
        _pos_chunk(pr1_v, e1_v, offs_v, idx1_v, j, nchunk)
        _pos_chunk(pr2_v, e2_v, offs_v, idx2_v, j, nchunk)
        c1 = pltpu.async_copy(rows_v, xs_hbm.at[idx1_v], sem)
        c2 = pltpu.async_copy(rows_v, xs_hbm.at[idx2_v], sem)
        c1.wait()
        c2.wait()


def _run_dispatch(xpk, pr1m, pr2m, e1m, e2m, offs16, t, h, e):
    nw = 32
    tok_per_w = t // nw
    nchunk = 32
    hp = h // 2
    mesh = plsc.VectorSubcoreMesh(core_axis_name="c", subcore_axis_name="s")
    body = functools.partial(_dispatch_body, tok_per_w=tok_per_w,
                             nchunk=nchunk)
    return pl.kernel(
        body,
        out_type=jax.ShapeDtypeStruct((TOPK * t, hp), jnp.int32),
        mesh=mesh,
        compiler_params=pltpu.CompilerParams(needs_layout_passes=False),
        scratch_types=[
            pltpu.VMEM((e,), jnp.float32),
            pltpu.VMEM((tok_per_w,), jnp.float32),
            pltpu.VMEM((tok_per_w,), jnp.float32),
            pltpu.VMEM((tok_per_w,), jnp.float32),
            pltpu.VMEM((tok_per_w,), jnp.float32),
            pltpu.VMEM((nchunk,), jnp.int32),
            pltpu.VMEM((nchunk,), jnp.int32),
            pltpu.VMEM((nchunk, hp), jnp.int32),
            pltpu.SemaphoreType.DMA,
        ],
    )(xpk, pr1m, pr2m, e1m, e2m, offs16)


def _group_body(offs_ref, xs_ref, gwl_ref, gwh_ref, dw_ref, o_ref, acc_ref,
                *, n_experts, bm):
    i = pl.program_id(0)
    lo, hi = _unpack_pair(xs_ref[...])
    xl = lo.astype(jnp.bfloat16)
    xh = hi.astype(jnp.bfloat16)
    row0 = i * bm
    row_g = row0 + lax.broadcasted_iota(jnp.int32, (bm, 1), 0)
    acc_ref[...] = jnp.zeros(acc_ref.shape, jnp.float32)
    for ex in range(n_experts):
        start = offs_ref[ex]
        end = offs_ref[ex + 1]
        cond = (start < row0 + bm) & (end > row0)

        @pl.when(cond)
        def _(ex=ex, start=start, end=end):
            hp = lax.dot_general(xl, gwl_ref[ex], (((1,), (1,)), ((), ())),
                                 preferred_element_type=jnp.float32)
            hp += lax.dot_general(xh, gwh_ref[ex], (((1,), (1,)), ((), ())),
                                  preferred_element_type=jnp.float32)
            a = hp * jax.nn.sigmoid(hp)
            pe = lax.dot_general(a.astype(jnp.bfloat16), dw_ref[ex],
                                 (((1,), (1,)), ((), ())),
                                 preferred_element_type=jnp.float32)
            mask = (row_g >= start) & (row_g < end)
            acc_ref[...] += jnp.where(mask, pe, 0.0)
    o_ref[...] = _pack_pair(acc_ref[...])


def _run_grouped(offs_i, xs, gwl, gwh, down_bf, t2, h, f, e):
    bm = 256
    body = functools.partial(_group_body, n_experts=e, bm=bm)
    grid_spec = pltpu.PrefetchScalarGridSpec(
        num_scalar_prefetch=1,
        grid=(t2 // bm,),
        in_specs=[
            pl.BlockSpec((bm, h // 2), lambda i, offs: (i, 0)),
            pl.BlockSpec((e, f, h // 2), lambda i, offs: (0, 0, 0)),
            pl.BlockSpec((e, f, h // 2), lambda i, offs: (0, 0, 0)),
            pl.BlockSpec((e, h, f), lambda i, offs: (0, 0, 0)),
        ],
        out_specs=pl.BlockSpec((bm, h // 2), lambda i, offs: (i, 0)),
        scratch_shapes=[pltpu.VMEM((bm, h), jnp.float32)],
    )
    return pl.pallas_call(
        body,
        grid_spec=grid_spec,
        out_shape=jax.ShapeDtypeStruct((t2, h // 2), jnp.int32),
        compiler_params=pltpu.CompilerParams(
            dimension_semantics=("arbitrary",)),
    )(offs_i, xs, gwl, gwh, down_bf)


def _gather_body(eout_hbm, pr1_hbm, pr2_hbm, e1_hbm, e2_hbm, offs_hbm,
                 g1_hbm, g2_hbm,
                 offs_v, pr1_v, pr2_v, e1_v, e2_v, idx1_v, idx2_v,
                 g1_v, g2_v, sem, *, tok_per_w, nchunk):
    wid = lax.axis_index("s") * 2 + lax.axis_index("c")
    base = wid * tok_per_w
    pltpu.sync_copy(offs_hbm, offs_v)
    _load_meta_row(pr1_hbm, wid, pr1_v)
    _load_meta_row(pr2_hbm, wid, pr2_v)
    _load_meta_row(e1_hbm, wid, e1_v)
    _load_meta_row(e2_hbm, wid, e2_v)
    for j in range(tok_per_w // nchunk):
        tb = base + j * nchunk
        _pos_chunk(pr1_v, e1_v, offs_v, idx1_v, j, nchunk)
        _pos_chunk(pr2_v, e2_v, offs_v, idx2_v, j, nchunk)
        c1 = pltpu.async_copy(eout_hbm.at[idx1_v], g1_v, sem)
        c2 = pltpu.async_copy(eout_hbm.at[idx2_v], g2_v, sem)
        c1.wait()
        c2.wait()
        pltpu.sync_copy(g1_v, g1_hbm.at[pl.ds(tb, nchunk)])
        pltpu.sync_copy(g2_v, g2_hbm.at[pl.ds(tb, nchunk)])


def _run_gather(eout, pr1m, pr2m, e1m, e2m, offs16, t, h, e):
    nw = 32
    tok_per_w = t // nw
    nchunk = 32
    hp = h // 2
    mesh = plsc.VectorSubcoreMesh(core_axis_name="c", subcore_axis_name="s")
    body = functools.partial(_gather_body, tok_per_w=tok_per_w,
                             nchunk=nchunk)
    return pl.kernel(
        body,
        out_type=[jax.ShapeDtypeStruct((t, hp), jnp.int32)] * 2,
        mesh=mesh,
        compiler_params=pltpu.CompilerParams(needs_layout_passes=False),
        scratch_types=[
            pltpu.VMEM((e,), jnp.float32),
            pltpu.VMEM((tok_per_w,), jnp.float32),
            pltpu.VMEM((tok_per_w,), jnp.float32),
            pltpu.VMEM((tok_per_w,), jnp.float32),
            pltpu.VMEM((tok_per_w,), jnp.float32),
            pltpu.VMEM((nchunk,), jnp.int32),
            pltpu.VMEM((nchunk,), jnp.int32),
            pltpu.VMEM((nchunk, hp), jnp.int32),
            pltpu.VMEM((nchunk, hp), jnp.int32),
            pltpu.SemaphoreType.DMA,
        ],
    )(eout, pr1m, pr2m, e1m, e2m, offs16)


def _final_body(sh_ref, g1_ref, g2_ref, w1_ref, w2_ref, o_ref):
    h2 = g1_ref.shape[1]
    lo1, hi1 = _unpack_pair(g1_ref[...])
    lo2, hi2 = _unpack_pair(g2_ref[...])
    sh = sh_ref[...]
    w1 = w1_ref[...]
    w2 = w2_ref[...]
    o_ref[:, :h2] = sh[:, :h2] + w1 * lo1 + w2 * lo2
    o_ref[:, h2:] = sh[:, h2:] + w1 * hi1 + w2 * hi2


def _run_final(shared, g1, g2, w1c, w2c, t, h):
    tb = 512
    col = pl.BlockSpec((tb, 1), lambda i: (i, 0))
    half = pl.BlockSpec((tb, h // 2), lambda i: (i, 0))
    full = pl.BlockSpec((tb, h), lambda i: (i, 0))
    return pl.pallas_call(
        _final_body,
        grid=(t // tb,),
        in_specs=[full, half, half, col, col],
        out_specs=full,
        out_shape=jax.ShapeDtypeStruct((t, h), jnp.float32),
        compiler_params=pltpu.CompilerParams(
            dimension_semantics=("arbitrary",)),
    )(shared, g1, g2, w1c, w2c)


def kernel(hidden_states, router_w, expert_bias, gate_w, down_w,
           shared_gate_w, shared_down_w):
    b, s, h = hidden_states.shape
    t = b * s
    e, f, _ = gate_w.shape
    t2 = TOPK * t

    x = hidden_states.reshape(t, h)
    bias2d = expert_bias.reshape(1, e)
    gwl = gate_w[:, :, :h // 2].astype(jnp.bfloat16)
    gwh = gate_w[:, :, h // 2:].astype(jnp.bfloat16)
    down_bf = down_w.astype(jnp.bfloat16)
    sgw_bf = shared_gate_w.astype(jnp.bfloat16)
    sdw_bf = shared_down_w.astype(jnp.bfloat16)

    pr1m, pr2m, e1m, e2m, w1c, w2c, xpk, shared, offs = _run_router(
        x, router_w, bias2d, sgw_bf, sdw_bf, t, h, f, e)
    offs16 = offs.reshape(e)
    offs_i = jnp.concatenate(
        [offs16.astype(jnp.int32), jnp.array([t2], jnp.int32)])

    xs = _run_dispatch(xpk, pr1m, pr2m, e1m, e2m, offs16, t, h, e)
    eout = _run_grouped(offs_i, xs, gwl, gwh, down_bf, t2, h, f, e)
    g1, g2 = _run_gather(eout, pr1m, pr2m, e1m, e2m, offs16, t, h, e)
    final = _run_final(shared, g1, g2, w1c, w2c, t, h)
    return final.reshape(b, s, h)


# speedup vs baseline: 1.0466x; 1.0466x over previous
import functools

import jax
import jax.numpy as jnp
from jax import lax
from jax.experimental import pallas as pl
from jax.experimental.pallas import tpu as pltpu
from jax.experimental.pallas import tpu_sc as plsc

TOPK = 2
SCALE = 1.0

TB_R = 256


def _pack_pair(v32):
    h2 = v32.shape[1] // 2
    vi = lax.bitcast_convert_type(v32, jnp.int32) + 32768
    return ((vi[:, :h2] >> 16) & 65535) | (vi[:, h2:] & (-65536))


def _unpack_pair(w):
    lo = lax.bitcast_convert_type(w << 16, jnp.float32)
    hi = lax.bitcast_convert_type(w & (-65536), jnp.float32)
    return lo, hi


def _router_body(x_ref, rw_ref, b_ref,
                 pr1_ref, pr2_ref, e1_ref, e2_ref, w1c_ref, w2c_ref,
                 xpk_ref, offs_ref, hist_ref, *, n_experts):
    i = pl.program_id(0)
    e = n_experts

    @pl.when(i == 0)
    def _():
        hist_ref[...] = jnp.zeros((1, e), jnp.float32)

    x = x_ref[...]
    logits = lax.dot_general(x, rw_ref[...], (((1,), (1,)), ((), ())),
                             preferred_element_type=jnp.float32) + b_ref[...]
    probs = jax.nn.sigmoid(logits)
    iota = lax.broadcasted_iota(jnp.int32, (TB_R, e), 1)
    m1 = jnp.max(probs, axis=1, keepdims=True)
    idx1 = jnp.min(jnp.where(probs >= m1, iota, e), axis=1, keepdims=True)
    sel1 = iota == idx1
    probs2 = jnp.where(sel1, -jnp.inf, probs)
    m2 = jnp.max(probs2, axis=1, keepdims=True)
    idx2 = jnp.min(jnp.where(probs2 >= m2, iota, e), axis=1, keepdims=True)
    sel2 = iota == idx2
    denom = m1 + m2 + 1e-9
    w1 = m1 / denom * SCALE
    w2 = m2 / denom * SCALE

    oh = jnp.where(sel1 | sel2, 1.0, 0.0)
    r_i = lax.broadcasted_iota(jnp.int32, (TB_R, TB_R), 0)
    c_i = lax.broadcasted_iota(jnp.int32, (TB_R, TB_R), 1)
    lstrict = jnp.where(c_i < r_i, 1.0, 0.0)
    excl = lax.dot_general(lstrict, oh, (((1,), (0,)), ((), ())),
                           preferred_element_type=jnp.float32,
                           precision=lax.Precision.HIGHEST)
    excl = excl + hist_ref[...]
    pr1 = jnp.sum(excl * jnp.where(sel1, 1.0, 0.0), axis=1, keepdims=True)
    pr2 = jnp.sum(excl * jnp.where(sel2, 1.0, 0.0), axis=1, keepdims=True)

    hist_new = hist_ref[...] + jnp.sum(oh, axis=0, keepdims=True)
    hist_ref[...] = hist_new

    re_i = lax.broadcasted_iota(jnp.int32, (e, e), 0)
    ce_i = lax.broadcasted_iota(jnp.int32, (e, e), 1)
    le = jnp.where(re_i < ce_i, 1.0, 0.0)
    offs = lax.dot_general(hist_new, le, (((1,), (0,)), ((), ())),
                           preferred_element_type=jnp.float32,
                           precision=lax.Precision.HIGHEST)
    offs_ref[...] = offs.reshape(1, 1, e)

    ident = jnp.where(r_i[:128, :128] == c_i[:128, :128], 1.0, 0.0)

    def t(col):
        top = lax.dot_general(col[0:128, :], ident, (((0,), (0,)), ((), ())),
                              preferred_element_type=jnp.float32,
                              precision=lax.Precision.HIGHEST)
        bot = lax.dot_general(col[128:256, :], ident, (((0,), (0,)), ((), ())),
                              preferred_element_type=jnp.float32,
                              precision=lax.Precision.HIGHEST)
        return jnp.concatenate([top, bot], axis=0).reshape(1, 2, 128)

    pr1_ref[...] = t(pr1)
    pr2_ref[...] = t(pr2)
    e1_ref[...] = t(idx1.astype(jnp.float32))
    e2_ref[...] = t(idx2.astype(jnp.float32))
    w1c_ref[...] = w1
    w2c_ref[...] = w2
    xpk_ref[...] = _pack_pair(x)


def _run_router(x, router_w, bias2d, t, h, e):
    nb = t // TB_R
    meta_shape = jax.ShapeDtypeStruct((t // 256, 2, 128), jnp.float32)
    body = functools.partial(_router_body, n_experts=e)
    cmeta = pl.BlockSpec((1, 2, 128), lambda i: (i, 0, 0))
    return pl.pallas_call(
        body,
        grid=(nb,),
        in_specs=[
            pl.BlockSpec((TB_R, h), lambda i: (i, 0)),
            pl.BlockSpec((e, h), lambda i: (0, 0)),
            pl.BlockSpec((1, e), lambda i: (0, 0)),
        ],
        out_specs=[cmeta, cmeta, cmeta, cmeta,
                   pl.BlockSpec((TB_R, 1), lambda i: (i, 0)),
                   pl.BlockSpec((TB_R, 1), lambda i: (i, 0)),
                   pl.BlockSpec((TB_R, h // 2), lambda i: (i, 0)),
                   pl.BlockSpec((1, 1, e), lambda i: (0, 0, 0))],
        out_shape=[meta_shape] * 4 + [
            jax.ShapeDtypeStruct((t, 1), jnp.float32)] * 2 + [
            jax.ShapeDtypeStruct((t, h // 2), jnp.int32),
            jax.ShapeDtypeStruct((1, 1, e), jnp.float32)],
        scratch_shapes=[pltpu.VMEM((1, e), jnp.float32)],
        compiler_params=pltpu.CompilerParams(
            dimension_semantics=("arbitrary",)),
    )(x, router_w, bias2d)


def _ffn_bf16(xb, gw, dw):
    hp = lax.dot_general(xb, gw, (((1,), (1,)), ((), ())),
                         preferred_element_type=jnp.float32)
    a = hp * jax.nn.sigmoid(hp)
    return lax.dot_general(a.astype(jnp.bfloat16), dw,
                           (((1,), (1,)), ((), ())),
                           preferred_element_type=jnp.float32)


def _shared_body(x_ref, sgw_ref, sdw_ref, o_ref):
    o_ref[...] = _ffn_bf16(x_ref[...].astype(jnp.bfloat16),
                           sgw_ref[...], sdw_ref[...])


def _run_shared(x, sgw_bf, sdw_bf, t, h, f):
    tb = 512
    return pl.pallas_call(
        _shared_body,
        grid=(t // tb,),
        in_specs=[
            pl.BlockSpec((tb, h), lambda i: (i, 0)),
            pl.BlockSpec((f, h), lambda i: (0, 0)),
            pl.BlockSpec((h, f), lambda i: (0, 0)),
        ],
        out_specs=pl.BlockSpec((tb, h), lambda i: (i, 0)),
        out_shape=jax.ShapeDtypeStruct((t, h), jnp.float32),
        compiler_params=pltpu.CompilerParams(
            dimension_semantics=("arbitrary",)),
    )(x, sgw_bf, sdw_bf)


def _load_meta_row(hbm, wid, dst):
    pltpu.sync_copy(hbm.at[wid // 2, wid % 2], dst)


def _pos_chunk(pr_v, e_v, offs_v, idx_v, j, nchunk):
    for c in range(nchunk // 16):
        sl = pl.ds(j * nchunk + c * 16, 16)
        eidx = e_v[sl].astype(jnp.int32)
        ofs = plsc.load_gather(offs_v, [eidx])
        idx_v[pl.ds(c * 16, 16)] = (pr_v[sl] + ofs + 0.5).astype(jnp.int32)


def _dispatch_body(x_hbm, pr1_hbm, pr2_hbm, e1_hbm, e2_hbm, offs_hbm,
                   xs_hbm,
                   offs_v, pr1_v, pr2_v, e1_v, e2_v, idx1_v, idx2_v,
                   rows_v, sem, *, tok_per_w, nchunk):
    wid = lax.axis_index("s") * 2 + lax.axis_index("c")
    base = wid * tok_per_w
    pltpu.sync_copy(offs_hbm, offs_v)
    _load_meta_row(pr1_hbm, wid, pr1_v)
    _load_meta_row(pr2_hbm, wid, pr2_v)
    _load_meta_row(e1_hbm, wid, e1_v)
    _load_meta_row(e2_hbm, wid, e2_v)
    for j in range(tok_per_w // nchunk):
        pltpu.sync_copy(x_hbm.at[pl.ds(base + j * nchunk, nchunk)], rows_v)
        _pos_chunk(pr1_v, e1_v, offs_v, idx1_v, j, nchunk)
        _pos_chunk(pr2_v, e2_v, offs_v, idx2_v, j, nchunk)
        c1 = pltpu.async_copy(rows_v, xs_hbm.at[idx1_v], sem)
        c2 = pltpu.async_copy(rows_v, xs_hbm.at[idx2_v], sem)
        c1.wait()
        c2.wait()


def _run_dispatch(xpk, pr1m, pr2m, e1m, e2m, offs16, t, h, e):
    nw = 32
    tok_per_w = t // nw
    nchunk = 64
    hp = h // 2
    mesh = plsc.VectorSubcoreMesh(core_axis_name="c", subcore_axis_name="s")
    body = functools.partial(_dispatch_body, tok_per_w=tok_per_w,
                             nchunk=nchunk)
    return pl.kernel(
        body,
        out_type=jax.ShapeDtypeStruct((TOPK * t, hp), jnp.int32),
        mesh=mesh,
        compiler_params=pltpu.CompilerParams(needs_layout_passes=False),
        scratch_types=[
            pltpu.VMEM((e,), jnp.float32),
            pltpu.VMEM((tok_per_w,), jnp.float32),
            pltpu.VMEM((tok_per_w,), jnp.float32),
            pltpu.VMEM((tok_per_w,), jnp.float32),
            pltpu.VMEM((tok_per_w,), jnp.float32),
            pltpu.VMEM((nchunk,), jnp.int32),
            pltpu.VMEM((nchunk,), jnp.int32),
            pltpu.VMEM((nchunk, hp), jnp.int32),
            pltpu.SemaphoreType.DMA,
        ],
    )(xpk, pr1m, pr2m, e1m, e2m, offs16)


def _group_body(offs_ref, xs_ref, gwl_ref, gwh_ref, dw_ref, o_ref, acc_ref,
                *, n_experts, bm):
    i = pl.program_id(0)
    lo, hi = _unpack_pair(xs_ref[...])
    xl = lo.astype(jnp.bfloat16)
    xh = hi.astype(jnp.bfloat16)
    row0 = i * bm
    row_g = row0 + lax.broadcasted_iota(jnp.int32, (bm, 1), 0)
    acc_ref[...] = jnp.zeros(acc_ref.shape, jnp.float32)
    for ex in range(n_experts):
        start = offs_ref[ex]
        end = offs_ref[ex + 1]
        cond = (start < row0 + bm) & (end > row0)

        @pl.when(cond)
        def _(ex=ex, start=start, end=end):
            hp = lax.dot_general(xl, gwl_ref[ex], (((1,), (1,)), ((), ())),
                                 preferred_element_type=jnp.float32)
            hp += lax.dot_general(xh, gwh_ref[ex], (((1,), (1,)), ((), ())),
                                  preferred_element_type=jnp.float32)
            a = hp * jax.nn.sigmoid(hp)
            pe = lax.dot_general(a.astype(jnp.bfloat16), dw_ref[ex],
                                 (((1,), (1,)), ((), ())),
                                 preferred_element_type=jnp.float32)
            mask = (row_g >= start) & (row_g < end)
            acc_ref[...] += jnp.where(mask, pe, 0.0)
    o_ref[...] = _pack_pair(acc_ref[...])


def _run_grouped(offs_i, xs, gwl, gwh, down_bf, t2, h, f, e):
    bm = 256
    body = functools.partial(_group_body, n_experts=e, bm=bm)
    grid_spec = pltpu.PrefetchScalarGridSpec(
        num_scalar_prefetch=1,
        grid=(t2 // bm,),
        in_specs=[
            pl.BlockSpec((bm, h // 2), lambda i, offs: (i, 0)),
            pl.BlockSpec((e, f, h // 2), lambda i, offs: (0, 0, 0)),
            pl.BlockSpec((e, f, h // 2), lambda i, offs: (0, 0, 0)),
            pl.BlockSpec((e, h, f), lambda i, offs: (0, 0, 0)),
        ],
        out_specs=pl.BlockSpec((bm, h // 2), lambda i, offs: (i, 0)),
        scratch_shapes=[pltpu.VMEM((bm, h), jnp.float32)],
    )
    return pl.pallas_call(
        body,
        grid_spec=grid_spec,
        out_shape=jax.ShapeDtypeStruct((t2, h // 2), jnp.int32),
        compiler_params=pltpu.CompilerParams(
            dimension_semantics=("arbitrary",)),
    )(offs_i, xs, gwl, gwh, down_bf)


def _gather_body(eout_hbm, pr1_hbm, pr2_hbm, e1_hbm, e2_hbm, offs_hbm,
                 g1_hbm, g2_hbm,
                 offs_v, pr1_v, pr2_v, e1_v, e2_v, idx1_v, idx2_v,
                 g1_v, g2_v, sem, *, tok_per_w, nchunk):
    wid = lax.axis_index("s") * 2 + lax.axis_index("c")
    base = wid * tok_per_w
    pltpu.sync_copy(offs_hbm, offs_v)
    _load_meta_row(pr1_hbm, wid, pr1_v)
    _load_meta_row(pr2_hbm, wid, pr2_v)
    _load_meta_row(e1_hbm, wid, e1_v)
    _load_meta_row(e2_hbm, wid, e2_v)
    for j in range(tok_per_w // nchunk):
        tb = base + j * nchunk
        _pos_chunk(pr1_v, e1_v, offs_v, idx1_v, j, nchunk)
        _pos_chunk(pr2_v, e2_v, offs_v, idx2_v, j, nchunk)
        c1 = pltpu.async_copy(eout_hbm.at[idx1_v], g1_v, sem)
        c2 = pltpu.async_copy(eout_hbm.at[idx2_v], g2_v, sem)
        c1.wait()
        c2.wait()
        pltpu.sync_copy(g1_v, g1_hbm.at[pl.ds(tb, nchunk)])
        pltpu.sync_copy(g2_v, g2_hbm.at[pl.ds(tb, nchunk)])


def _run_gather(eout, pr1m, pr2m, e1m, e2m, offs16, t, h, e):
    nw = 32
    tok_per_w = t // nw
    nchunk = 64
    hp = h // 2
    mesh = plsc.VectorSubcoreMesh(core_axis_name="c", subcore_axis_name="s")
    body = functools.partial(_gather_body, tok_per_w=tok_per_w,
                             nchunk=nchunk)
    return pl.kernel(
        body,
        out_type=[jax.ShapeDtypeStruct((t, hp), jnp.int32)] * 2,
        mesh=mesh,
        compiler_params=pltpu.CompilerParams(needs_layout_passes=False),
        scratch_types=[
            pltpu.VMEM((e,), jnp.float32),
            pltpu.VMEM((tok_per_w,), jnp.float32),
            pltpu.VMEM((tok_per_w,), jnp.float32),
            pltpu.VMEM((tok_per_w,), jnp.float32),
            pltpu.VMEM((tok_per_w,), jnp.float32),
            pltpu.VMEM((nchunk,), jnp.int32),
            pltpu.VMEM((nchunk,), jnp.int32),
            pltpu.VMEM((nchunk, hp), jnp.int32),
            pltpu.VMEM((nchunk, hp), jnp.int32),
            pltpu.SemaphoreType.DMA,
        ],
    )(eout, pr1m, pr2m, e1m, e2m, offs16)


def _final_body(sh_ref, g1_ref, g2_ref, w1_ref, w2_ref, o_ref):
    h2 = g1_ref.shape[1]
    lo1, hi1 = _unpack_pair(g1_ref[...])
    lo2, hi2 = _unpack_pair(g2_ref[...])
    sh = sh_ref[...]
    w1 = w1_ref[...]
    w2 = w2_ref[...]
    o_ref[:, :h2] = sh[:, :h2] + w1 * lo1 + w2 * lo2
    o_ref[:, h2:] = sh[:, h2:] + w1 * hi1 + w2 * hi2


def _run_final(shared, g1, g2, w1c, w2c, t, h):
    tb = 512
    col = pl.BlockSpec((tb, 1), lambda i: (i, 0))
    half = pl.BlockSpec((tb, h // 2), lambda i: (i, 0))
    full = pl.BlockSpec((tb, h), lambda i: (i, 0))
    return pl.pallas_call(
        _final_body,
        grid=(t // tb,),
        in_specs=[full, half, half, col, col],
        out_specs=full,
        out_shape=jax.ShapeDtypeStruct((t, h), jnp.float32),
        compiler_params=pltpu.CompilerParams(
            dimension_semantics=("arbitrary",)),
    )(shared, g1, g2, w1c, w2c)


def kernel(hidden_states, router_w, expert_bias, gate_w, down_w,
           shared_gate_w, shared_down_w):
    b, s, h = hidden_states.shape
    t = b * s
    e, f, _ = gate_w.shape
    t2 = TOPK * t

    x = hidden_states.reshape(t, h)
    bias2d = expert_bias.reshape(1, e)
    gwl = gate_w[:, :, :h // 2].astype(jnp.bfloat16)
    gwh = gate_w[:, :, h // 2:].astype(jnp.bfloat16)
    down_bf = down_w.astype(jnp.bfloat16)
    sgw_bf = shared_gate_w.astype(jnp.bfloat16)
    sdw_bf = shared_down_w.astype(jnp.bfloat16)

    pr1m, pr2m, e1m, e2m, w1c, w2c, xpk, offs = _run_router(
        x, router_w, bias2d, t, h, e)
    offs16 = offs.reshape(e)
    offs_i = jnp.concatenate(
        [offs16.astype(jnp.int32), jnp.array([t2], jnp.int32)])

    shared = _run_shared(x, sgw_bf, sdw_bf, t, h, f)
    xs = _run_dispatch(xpk, pr1m, pr2m, e1m, e2m, offs16, t, h, e)
    eout = _run_grouped(offs_i, xs, gwl, gwh, down_bf, t2, h, f, e)
    g1, g2 = _run_gather(eout, pr1m, pr2m, e1m, e2m, offs16, t, h, e)
    final = _run_final(shared, g1, g2, w1c, w2c, t, h)
    return final.reshape(b, s, h)


# speedup vs baseline: 1.0479x; 1.0012x over previous
import functools

import jax
import jax.numpy as jnp
from jax import lax
from jax.experimental import pallas as pl
from jax.experimental.pallas import tpu as pltpu
from jax.experimental.pallas import tpu_sc as plsc

TOPK = 2
SCALE = 1.0

TB_R = 256


def _pack_pair(v32):
    h2 = v32.shape[1] // 2
    vi = lax.bitcast_convert_type(v32, jnp.int32) + 32768
    return ((vi[:, :h2] >> 16) & 65535) | (vi[:, h2:] & (-65536))


def _unpack_pair(w):
    lo = lax.bitcast_convert_type(w << 16, jnp.float32)
    hi = lax.bitcast_convert_type(w & (-65536), jnp.float32)
    return lo, hi


def _router_body(x_ref, rw_ref, b_ref,
                 pr1_ref, pr2_ref, e1_ref, e2_ref, w1c_ref, w2c_ref,
                 xpk_ref, offs_ref, hist_ref, *, n_experts):
    i = pl.program_id(0)
    e = n_experts

    @pl.when(i == 0)
    def _():
        hist_ref[...] = jnp.zeros((1, e), jnp.float32)

    x = x_ref[...]
    logits = lax.dot_general(x, rw_ref[...], (((1,), (1,)), ((), ())),
                             preferred_element_type=jnp.float32) + b_ref[...]
    probs = jax.nn.sigmoid(logits)
    iota = lax.broadcasted_iota(jnp.int32, (TB_R, e), 1)
    m1 = jnp.max(probs, axis=1, keepdims=True)
    idx1 = jnp.min(jnp.where(probs >= m1, iota, e), axis=1, keepdims=True)
    sel1 = iota == idx1
    probs2 = jnp.where(sel1, -jnp.inf, probs)
    m2 = jnp.max(probs2, axis=1, keepdims=True)
    idx2 = jnp.min(jnp.where(probs2 >= m2, iota, e), axis=1, keepdims=True)
    sel2 = iota == idx2
    denom = m1 + m2 + 1e-9
    w1 = m1 / denom * SCALE
    w2 = m2 / denom * SCALE

    oh = jnp.where(sel1 | sel2, 1.0, 0.0)
    r_i = lax.broadcasted_iota(jnp.int32, (TB_R, TB_R), 0)
    c_i = lax.broadcasted_iota(jnp.int32, (TB_R, TB_R), 1)
    lstrict = jnp.where(c_i < r_i, 1.0, 0.0)
    excl = lax.dot_general(lstrict, oh, (((1,), (0,)), ((), ())),
                           preferred_element_type=jnp.float32,
                           precision=lax.Precision.HIGHEST)
    excl = excl + hist_ref[...]
    pr1 = jnp.sum(excl * jnp.where(sel1, 1.0, 0.0), axis=1, keepdims=True)
    pr2 = jnp.sum(excl * jnp.where(sel2, 1.0, 0.0), axis=1, keepdims=True)

    hist_new = hist_ref[...] + jnp.sum(oh, axis=0, keepdims=True)
    hist_ref[...] = hist_new

    re_i = lax.broadcasted_iota(jnp.int32, (e, e), 0)
    ce_i = lax.broadcasted_iota(jnp.int32, (e, e), 1)
    le = jnp.where(re_i < ce_i, 1.0, 0.0)
    offs = lax.dot_general(hist_new, le, (((1,), (0,)), ((), ())),
                           preferred_element_type=jnp.float32,
                           precision=lax.Precision.HIGHEST)
    offs_ref[...] = offs.reshape(1, 1, e)

    ident = jnp.where(r_i[:128, :128] == c_i[:128, :128], 1.0, 0.0)

    def t(col):
        top = lax.dot_general(col[0:128, :], ident, (((0,), (0,)), ((), ())),
                              preferred_element_type=jnp.float32,
                              precision=lax.Precision.HIGHEST)
        bot = lax.dot_general(col[128:256, :], ident, (((0,), (0,)), ((), ())),
                              preferred_element_type=jnp.float32,
                              precision=lax.Precision.HIGHEST)
        return jnp.concatenate([top, bot], axis=0).reshape(1, 2, 128)

    pr1_ref[...] = t(pr1)
    pr2_ref[...] = t(pr2)
    e1_ref[...] = t(idx1.astype(jnp.float32))
    e2_ref[...] = t(idx2.astype(jnp.float32))
    w1c_ref[...] = w1
    w2c_ref[...] = w2
    xpk_ref[...] = _pack_pair(x)


def _run_router(x, router_w, bias2d, t, h, e):
    nb = t // TB_R
    meta_shape = jax.ShapeDtypeStruct((t // 256, 2, 128), jnp.float32)
    body = functools.partial(_router_body, n_experts=e)
    cmeta = pl.BlockSpec((1, 2, 128), lambda i: (i, 0, 0))
    return pl.pallas_call(
        body,
        grid=(nb,),
        in_specs=[
            pl.BlockSpec((TB_R, h), lambda i: (i, 0)),
            pl.BlockSpec((e, h), lambda i: (0, 0)),
            pl.BlockSpec((1, e), lambda i: (0, 0)),
        ],
        out_specs=[cmeta, cmeta, cmeta, cmeta,
                   pl.BlockSpec((TB_R, 1), lambda i: (i, 0)),
                   pl.BlockSpec((TB_R, 1), lambda i: (i, 0)),
                   pl.BlockSpec((TB_R, h // 2), lambda i: (i, 0)),
                   pl.BlockSpec((1, 1, e), lambda i: (0, 0, 0))],
        out_shape=[meta_shape] * 4 + [
            jax.ShapeDtypeStruct((t, 1), jnp.float32)] * 2 + [
            jax.ShapeDtypeStruct((t, h // 2), jnp.int32),
            jax.ShapeDtypeStruct((1, 1, e), jnp.float32)],
        scratch_shapes=[pltpu.VMEM((1, e), jnp.float32)],
        compiler_params=pltpu.CompilerParams(
            dimension_semantics=("arbitrary",)),
    )(x, router_w, bias2d)


def _ffn_bf16(xb, gw, dw):
    hp = lax.dot_general(xb, gw, (((1,), (1,)), ((), ())),
                         preferred_element_type=jnp.float32)
    a = hp * jax.nn.sigmoid(hp)
    return lax.dot_general(a.astype(jnp.bfloat16), dw,
                           (((1,), (1,)), ((), ())),
                           preferred_element_type=jnp.float32)


def _shared_body(x_ref, sgw_ref, sdw_ref, o_ref):
    o_ref[...] = _ffn_bf16(x_ref[...].astype(jnp.bfloat16),
                           sgw_ref[...], sdw_ref[...])


def _run_shared(x, sgw_bf, sdw_bf, t, h, f):
    tb = 512
    return pl.pallas_call(
        _shared_body,
        grid=(t // tb,),
        in_specs=[
            pl.BlockSpec((tb, h), lambda i: (i, 0)),
            pl.BlockSpec((f, h), lambda i: (0, 0)),
            pl.BlockSpec((h, f), lambda i: (0, 0)),
        ],
        out_specs=pl.BlockSpec((tb, h), lambda i: (i, 0)),
        out_shape=jax.ShapeDtypeStruct((t, h), jnp.float32),
        compiler_params=pltpu.CompilerParams(
            dimension_semantics=("arbitrary",)),
    )(x, sgw_bf, sdw_bf)


def _load_meta_row(hbm, wid, dst):
    pltpu.sync_copy(hbm.at[wid // 2, wid % 2], dst)


def _pos_chunk(pr_v, e_v, offs_v, idx_v, j, nchunk):
    for c in range(nchunk // 16):
        sl = pl.ds(j * nchunk + c * 16, 16)
        eidx = e_v[sl].astype(jnp.int32)
        ofs = plsc.load_gather(offs_v, [eidx])
        idx_v[pl.ds(c * 16, 16)] = (pr_v[sl] + ofs + 0.5).astype(jnp.int32)


def _dispatch_body(x_hbm, pr1_hbm, pr2_hbm, e1_hbm, e2_hbm, offs_hbm,
                   xs_hbm,
                   offs_v, pr1_v, pr2_v, e1_v, e2_v, idx1_v, idx2_v,
                   rows_v, sem, *, tok_per_w, nchunk):
    wid = lax.axis_index("s") * 2 + lax.axis_index("c")
    base = wid * tok_per_w
    pltpu.sync_copy(offs_hbm, offs_v)
    _load_meta_row(pr1_hbm, wid, pr1_v)
    _load_meta_row(pr2_hbm, wid, pr2_v)
    _load_meta_row(e1_hbm, wid, e1_v)
    _load_meta_row(e2_hbm, wid, e2_v)
    for j in range(tok_per_w // nchunk):
        pltpu.sync_copy(x_hbm.at[pl.ds(base + j * nchunk, nchunk)], rows_v)
        _pos_chunk(pr1_v, e1_v, offs_v, idx1_v, j, nchunk)
        _pos_chunk(pr2_v, e2_v, offs_v, idx2_v, j, nchunk)
        c1 = pltpu.async_copy(rows_v, xs_hbm.at[idx1_v], sem)
        c2 = pltpu.async_copy(rows_v, xs_hbm.at[idx2_v], sem)
        c1.wait()
        c2.wait()


def _run_dispatch(xpk, pr1m, pr2m, e1m, e2m, offs16, t, h, e):
    nw = 32
    tok_per_w = t // nw
    nchunk = 128
    hp = h // 2
    mesh = plsc.VectorSubcoreMesh(core_axis_name="c", subcore_axis_name="s")
    body = functools.partial(_dispatch_body, tok_per_w=tok_per_w,
                             nchunk=nchunk)
    return pl.kernel(
        body,
        out_type=jax.ShapeDtypeStruct((TOPK * t, hp), jnp.int32),
        mesh=mesh,
        compiler_params=pltpu.CompilerParams(needs_layout_passes=False),
        scratch_types=[
            pltpu.VMEM((e,), jnp.float32),
            pltpu.VMEM((tok_per_w,), jnp.float32),
            pltpu.VMEM((tok_per_w,), jnp.float32),
            pltpu.VMEM((tok_per_w,), jnp.float32),
            pltpu.VMEM((tok_per_w,), jnp.float32),
            pltpu.VMEM((nchunk,), jnp.int32),
            pltpu.VMEM((nchunk,), jnp.int32),
            pltpu.VMEM((nchunk, hp), jnp.int32),
            pltpu.SemaphoreType.DMA,
        ],
    )(xpk, pr1m, pr2m, e1m, e2m, offs16)


def _group_body(offs_ref, xs_ref, gwl_ref, gwh_ref, dw_ref, o_ref, acc_ref,
                *, n_experts, bm):
    i = pl.program_id(0)
    lo, hi = _unpack_pair(xs_ref[...])
    xl = lo.astype(jnp.bfloat16)
    xh = hi.astype(jnp.bfloat16)
    row0 = i * bm
    row_g = row0 + lax.broadcasted_iota(jnp.int32, (bm, 1), 0)
    acc_ref[...] = jnp.zeros(acc_ref.shape, jnp.float32)
    for ex in range(n_experts):
        start = offs_ref[ex]
        end = offs_ref[ex + 1]
        cond = (start < row0 + bm) & (end > row0)

        @pl.when(cond)
        def _(ex=ex, start=start, end=end):
            hp = lax.dot_general(xl, gwl_ref[ex], (((1,), (1,)), ((), ())),
                                 preferred_element_type=jnp.float32)
            hp += lax.dot_general(xh, gwh_ref[ex], (((1,), (1,)), ((), ())),
                                  preferred_element_type=jnp.float32)
            a = hp * jax.nn.sigmoid(hp)
            pe = lax.dot_general(a.astype(jnp.bfloat16), dw_ref[ex],
                                 (((1,), (1,)), ((), ())),
                                 preferred_element_type=jnp.float32)
            mask = (row_g >= start) & (row_g < end)
            acc_ref[...] += jnp.where(mask, pe, 0.0)
    o_ref[...] = _pack_pair(acc_ref[...])


def _run_grouped(offs_i, xs, gwl, gwh, down_bf, t2, h, f, e):
    bm = 256
    body = functools.partial(_group_body, n_experts=e, bm=bm)
    grid_spec = pltpu.PrefetchScalarGridSpec(
        num_scalar_prefetch=1,
        grid=(t2 // bm,),
        in_specs=[
            pl.BlockSpec((bm, h // 2), lambda i, offs: (i, 0)),
            pl.BlockSpec((e, f, h // 2), lambda i, offs: (0, 0, 0)),
            pl.BlockSpec((e, f, h // 2), lambda i, offs: (0, 0, 0)),
            pl.BlockSpec((e, h, f), lambda i, offs: (0, 0, 0)),
        ],
        out_specs=pl.BlockSpec((bm, h // 2), lambda i, offs: (i, 0)),
        scratch_shapes=[pltpu.VMEM((bm, h), jnp.float32)],
    )
    return pl.pallas_call(
        body,
        grid_spec=grid_spec,
        out_shape=jax.ShapeDtypeStruct((t2, h // 2), jnp.int32),
        compiler_params=pltpu.CompilerParams(
            dimension_semantics=("arbitrary",)),
    )(offs_i, xs, gwl, gwh, down_bf)


def _gather_body(eout_hbm, pr1_hbm, pr2_hbm, e1_hbm, e2_hbm, offs_hbm,
                 g1_hbm, g2_hbm,
                 offs_v, pr1_v, pr2_v, e1_v, e2_v, idx1_v, idx2_v,
                 g1_v, g2_v, sem, *, tok_per_w, nchunk):
    wid = lax.axis_index("s") * 2 + lax.axis_index("c")
    base = wid * tok_per_w
    pltpu.sync_copy(offs_hbm, offs_v)
    _load_meta_row(pr1_hbm, wid, pr1_v)
    _load_meta_row(pr2_hbm, wid, pr2_v)
    _load_meta_row(e1_hbm, wid, e1_v)
    _load_meta_row(e2_hbm, wid, e2_v)
    for j in range(tok_per_w // nchunk):
        tb = base + j * nchunk
        _pos_chunk(pr1_v, e1_v, offs_v, idx1_v, j, nchunk)
        _pos_chunk(pr2_v, e2_v, offs_v, idx2_v, j, nchunk)
        c1 = pltpu.async_copy(eout_hbm.at[idx1_v], g1_v, sem)
        c2 = pltpu.async_copy(eout_hbm.at[idx2_v], g2_v, sem)
        c1.wait()
        c2.wait()
        pltpu.sync_copy(g1_v, g1_hbm.at[pl.ds(tb, nchunk)])
        pltpu.sync_copy(g2_v, g2_hbm.at[pl.ds(tb, nchunk)])


def _run_gather(eout, pr1m, pr2m, e1m, e2m, offs16, t, h, e):
    nw = 32
    tok_per_w = t // nw
    nchunk = 64
    hp = h // 2
    mesh = plsc.VectorSubcoreMesh(core_axis_name="c", subcore_axis_name="s")
    body = functools.partial(_gather_body, tok_per_w=tok_per_w,
                             nchunk=nchunk)
    return pl.kernel(
        body,
        out_type=[jax.ShapeDtypeStruct((t, hp), jnp.int32)] * 2,
        mesh=mesh,
        compiler_params=pltpu.CompilerParams(needs_layout_passes=False),
        scratch_types=[
            pltpu.VMEM((e,), jnp.float32),
            pltpu.VMEM((tok_per_w,), jnp.float32),
            pltpu.VMEM((tok_per_w,), jnp.float32),
            pltpu.VMEM((tok_per_w,), jnp.float32),
            pltpu.VMEM((tok_per_w,), jnp.float32),
            pltpu.VMEM((nchunk,), jnp.int32),
            pltpu.VMEM((nchunk,), jnp.int32),
            pltpu.VMEM((nchunk, hp), jnp.int32),
            pltpu.VMEM((nchunk, hp), jnp.int32),
            pltpu.SemaphoreType.DMA,
        ],
    )(eout, pr1m, pr2m, e1m, e2m, offs16)


def _final_body(sh_ref, g1_ref, g2_ref, w1_ref, w2_ref, o_ref):
    h2 = g1_ref.shape[1]
    lo1, hi1 = _unpack_pair(g1_ref[...])
    lo2, hi2 = _unpack_pair(g2_ref[...])
    sh = sh_ref[...]
    w1 = w1_ref[...]
    w2 = w2_ref[...]
    o_ref[:, :h2] = sh[:, :h2] + w1 * lo1 + w2 * lo2
    o_ref[:, h2:] = sh[:, h2:] + w1 * hi1 + w2 * hi2


def _run_final(shared, g1, g2, w1c, w2c, t, h):
    tb = 512
    col = pl.BlockSpec((tb, 1), lambda i: (i, 0))
    half = pl.BlockSpec((tb, h // 2), lambda i: (i, 0))
    full = pl.BlockSpec((tb, h), lambda i: (i, 0))
    return pl.pallas_call(
        _final_body,
        grid=(t // tb,),
        in_specs=[full, half, half, col, col],
        out_specs=full,
        out_shape=jax.ShapeDtypeStruct((t, h), jnp.float32),
        compiler_params=pltpu.CompilerParams(
            dimension_semantics=("arbitrary",)),
    )(shared, g1, g2, w1c, w2c)


def kernel(hidden_states, router_w, expert_bias, gate_w, down_w,
           shared_gate_w, shared_down_w):
    b, s, h = hidden_states.shape
    t = b * s
    e, f, _ = gate_w.shape
    t2 = TOPK * t

    x = hidden_states.reshape(t, h)
    bias2d = expert_bias.reshape(1, e)
    gwl = gate_w[:, :, :h // 2].astype(jnp.bfloat16)
    gwh = gate_w[:, :, h // 2:].astype(jnp.bfloat16)
    down_bf = down_w.astype(jnp.bfloat16)
    sgw_bf = shared_gate_w.astype(jnp.bfloat16)
    sdw_bf = shared_down_w.astype(jnp.bfloat16)

    pr1m, pr2m, e1m, e2m, w1c, w2c, xpk, offs = _run_router(
        x, router_w, bias2d, t, h, e)
    offs16 = offs.reshape(e)
    offs_i = jnp.concatenate(
        [offs16.astype(jnp.int32), jnp.array([t2], jnp.int32)])

    shared = _run_shared(x, sgw_bf, sdw_bf, t, h, f)
    xs = _run_dispatch(xpk, pr1m, pr2m, e1m, e2m, offs16, t, h, e)
    eout = _run_grouped(offs_i, xs, gwl, gwh, down_bf, t2, h, f, e)
    g1, g2 = _run_gather(eout, pr1m, pr2m, e1m, e2m, offs16, t, h, e)
    final = _run_final(shared, g1, g2, w1c, w2c, t, h)
    return final.reshape(b, s, h)
